# baseline (speedup 1.0000x reference)
import functools

import jax
import jax.numpy as jnp
from jax import lax
from jax.experimental import pallas as pl
from jax.experimental.pallas import tpu as pltpu
from jax.experimental.pallas import tpu_sc as plsc

VOCAB = 1000000
HID = 64
BATCH = 16384

NUM_CORES = 2
NUM_SUBCORES = 16
NUM_WORKERS = NUM_CORES * NUM_SUBCORES
NT = BATCH // NUM_WORKERS
S = 3
SPAN = 128 * S
NHB = HID // 16
NCOLS_PAD = (VOCAB + 127) // 128
MAX_C0 = NCOLS_PAD - S


def _sweep_body(meta, tabT, outT, sh_meta, smeta, blk0, blk1, blk2, out_v,
                s0, s1, s2):
    sid = lax.axis_index("s")
    wid = sid * NUM_CORES + lax.axis_index("c")
    base = wid * NT
    pltpu.sync_copy(meta.at[wid], sh_meta.at[sid])
    pltpu.sync_copy(sh_meta.at[sid], smeta)

    iota = lax.iota(jnp.int32, 16)
    c_start = smeta[0, 0] >> 7
    c_end = smeta[0, NT - 1] >> 7
    ngroups = (c_end - c_start) // S + 1
    niter = (ngroups + 2) // 3

    def cbase(g):
        return jnp.clip(c_start + g * S, 0, MAX_C0)

    def fire(g, blk, sem):
        return pltpu.async_copy(tabT.at[:, pl.ds(cbase(g) * 128, SPAN)],
                                blk, sem)

    def drain(blk, sem):
        pltpu.make_async_copy(tabT.at[:, pl.ds(0, SPAN)], blk, sem).wait()

    def extract_group(g, blk, t):
        lbase = cbase(g) << 7
        for j in range(S):
            c = c_start + g * S + j
            tc = jnp.minimum(t, NT - 1)
            hit = jnp.logical_and(t < NT, (smeta[0, tc] >> 7) == c)
            e = jnp.where(hit, smeta[1, tc], t)

            @pl.loop(t, e)
            def _tok(tt):
                lane = smeta[0, tt] - lbase
                lanes = jnp.full((16,), lane, jnp.int32)
                tvec = jnp.full((16,), tt, jnp.int32)
                for hb in range(NHB):
                    rows = iota + (hb * 16)
                    vals = plsc.load_gather(blk, [rows, lanes])
                    plsc.store_scatter(out_v, [rows, tvec], vals)

            t = e
        return t

    fire(0, blk0, s0)
    fire(1, blk1, s1)

    @pl.loop(0, niter, init_carry=0)
    def _group(i, t):
        g = 3 * i
        fire(g + 2, blk2, s2)
        drain(blk0, s0)
        t = extract_group(g, blk0, t)
        fire(g + 3, blk0, s0)
        drain(blk1, s1)
        t = extract_group(g + 1, blk1, t)
        fire(g + 4, blk1, s1)
        drain(blk2, s2)
        t = extract_group(g + 2, blk2, t)
        return t

    drain(blk0, s0)
    drain(blk1, s1)
    pltpu.sync_copy(out_v, outT.at[:, pl.ds(base, NT)])


@jax.jit
def _sweep(meta, tabT):
    mesh = plsc.VectorSubcoreMesh(core_axis_name="c", subcore_axis_name="s")
    kern = functools.partial(
        pl.kernel,
        mesh=mesh,
        out_type=jax.ShapeDtypeStruct((HID, BATCH), jnp.float32),
        scratch_types=[
            pltpu.VMEM_SHARED((NUM_SUBCORES, 2, NT), jnp.int32),
            pltpu.SMEM((2, NT), jnp.int32),
            pltpu.VMEM((HID, SPAN), jnp.float32),
            pltpu.VMEM((HID, SPAN), jnp.float32),
            pltpu.VMEM((HID, SPAN), jnp.float32),
            pltpu.VMEM((HID, NT), jnp.float32),
        ] + [pltpu.SemaphoreType.DMA] * 3,
        compiler_params=pltpu.CompilerParams(needs_layout_passes=False),
    )(_sweep_body)
    return kern(meta, tabT)


def kernel(x, embedding_table):
    tabT = embedding_table.T
    key = lax.sort(((x >> 7) << 14) | jnp.arange(BATCH, dtype=jnp.int32),
                   is_stable=False)
    perm = key & (BATCH - 1)
    xs = x[perm]
    cols = key >> 14
    nxt = jnp.concatenate([cols[1:], jnp.full((1,), -1, cols.dtype)])
    run_end = jnp.where(cols != nxt,
                        jnp.arange(1, BATCH + 1, dtype=jnp.int32),
                        jnp.int32(BATCH))
    ends_g = lax.cummin(run_end, axis=0, reverse=True)
    wbase = (jnp.arange(BATCH, dtype=jnp.int32) // NT) * NT
    ends_local = jnp.clip(ends_g - wbase, 0, NT)
    meta = jnp.stack([xs.reshape(NUM_WORKERS, NT),
                      ends_local.reshape(NUM_WORKERS, NT)], axis=1)
    outT = _sweep(meta, tabT)
    inv = jnp.zeros((BATCH,), jnp.int32).at[perm].set(
        jnp.arange(BATCH, dtype=jnp.int32))
    return jnp.take(outT.T, inv, axis=0)


# speedup vs baseline: 1.0263x; 1.0263x over previous
import functools

import jax
import jax.numpy as jnp
from jax import lax
from jax.experimental import pallas as pl
from jax.experimental.pallas import tpu as pltpu
from jax.experimental.pallas import tpu_sc as plsc

VOCAB = 1000000
HID = 64
BATCH = 16384

NUM_CORES = 2
NUM_SUBCORES = 16
NUM_WORKERS = NUM_CORES * NUM_SUBCORES
NT = BATCH // NUM_WORKERS
S = 3
SPAN = 128 * S
NHB = HID // 16
NCOLS_PAD = (VOCAB + 127) // 128
MAX_C0 = NCOLS_PAD - S


def _sweep_body(meta, tabT, outT, sh_meta, smeta, blk0, blk1, blk2, out_v,
                s0, s1, s2):
    sid = lax.axis_index("s")
    wid = sid * NUM_CORES + lax.axis_index("c")
    base = wid * NT
    pltpu.sync_copy(meta.at[wid], sh_meta.at[sid])
    pltpu.sync_copy(sh_meta.at[sid], smeta)

    iota = lax.iota(jnp.int32, 16)
    c_start = smeta[0, 0] >> 7
    c_end = smeta[0, NT - 1] >> 7
    ngroups = (c_end - c_start) // S + 1
    niter = (ngroups + 2) // 3

    def cbase(g):
        return jnp.clip(c_start + g * S, 0, MAX_C0)

    def fire(g, blk, sem):
        return pltpu.async_copy(tabT.at[:, pl.ds(cbase(g) * 128, SPAN)],
                                blk, sem)

    def drain(blk, sem):
        pltpu.make_async_copy(tabT.at[:, pl.ds(0, SPAN)], blk, sem).wait()

    def extract_group(g, blk, t):
        lbase = cbase(g) << 7
        for j in range(S):
            c = c_start + g * S + j
            tc = jnp.minimum(t, NT - 1)
            hit = jnp.logical_and(t < NT, (smeta[0, tc] >> 7) == c)
            e = jnp.where(hit, smeta[1, tc], t)

            @pl.loop(t, e)
            def _tok(tt):
                lane = smeta[0, tt] - lbase
                lanes = jnp.full((16,), lane, jnp.int32)
                tvec = jnp.full((16,), tt, jnp.int32)
                for hb in range(NHB):
                    rows = iota + (hb * 16)
                    vals = plsc.load_gather(blk, [rows, lanes])
                    plsc.store_scatter(out_v, [rows, tvec], vals)

            t = e
        return t

    fire(0, blk0, s0)
    fire(1, blk1, s1)

    @pl.loop(0, niter, init_carry=0)
    def _group(i, t):
        g = 3 * i
        fire(g + 2, blk2, s2)
        drain(blk0, s0)
        t = extract_group(g, blk0, t)
        fire(g + 3, blk0, s0)
        drain(blk1, s1)
        t = extract_group(g + 1, blk1, t)
        fire(g + 4, blk1, s1)
        drain(blk2, s2)
        t = extract_group(g + 2, blk2, t)
        return t

    drain(blk0, s0)
    drain(blk1, s1)
    pltpu.sync_copy(out_v, outT.at[:, pl.ds(base, NT)])


@jax.jit
def _sweep(meta, tabT):
    mesh = plsc.VectorSubcoreMesh(core_axis_name="c", subcore_axis_name="s")
    kern = functools.partial(
        pl.kernel,
        mesh=mesh,
        out_type=jax.ShapeDtypeStruct((HID, BATCH), jnp.float32),
        scratch_types=[
            pltpu.VMEM_SHARED((NUM_SUBCORES, 2, NT), jnp.int32),
            pltpu.SMEM((2, NT), jnp.int32),
            pltpu.VMEM((HID, SPAN), jnp.float32),
            pltpu.VMEM((HID, SPAN), jnp.float32),
            pltpu.VMEM((HID, SPAN), jnp.float32),
            pltpu.VMEM((HID, NT), jnp.float32),
        ] + [pltpu.SemaphoreType.DMA] * 3,
        compiler_params=pltpu.CompilerParams(needs_layout_passes=False),
    )(_sweep_body)
    return kern(meta, tabT)


def kernel(x, embedding_table):
    tabT = embedding_table.T
    xs, perm = lax.sort((x, jnp.arange(BATCH, dtype=jnp.int32)),
                        is_stable=False, num_keys=1)
    cols = xs >> 7
    nxt = jnp.concatenate([cols[1:], jnp.full((1,), -1, cols.dtype)])
    run_end = jnp.where(cols != nxt,
                        jnp.arange(1, BATCH + 1, dtype=jnp.int32),
                        jnp.int32(BATCH))
    ends_g = lax.cummin(run_end, axis=0, reverse=True)
    wbase = (jnp.arange(BATCH, dtype=jnp.int32) // NT) * NT
    ends_local = jnp.clip(ends_g - wbase, 0, NT)
    meta = jnp.stack([xs.reshape(NUM_WORKERS, NT),
                      ends_local.reshape(NUM_WORKERS, NT)], axis=1)
    outT = _sweep(meta, tabT)
    inv = jnp.zeros((BATCH,), jnp.int32).at[perm].set(
        jnp.arange(BATCH, dtype=jnp.int32))
    return jnp.take(outT.T, inv, axis=0)
